# baseline (speedup 1.0000x reference)
import jax
import jax.numpy as jnp
from jax import lax
from jax.experimental import pallas as pl

NX = 10
NE = 5
T = 1000
N_NODES = 1024

_NB = 65536
_SUB = 512


def _alpha(t):
    halfpi = 0.5 * jnp.pi
    s = 0.01
    return jnp.cos(halfpi * (t / T + s) / (1 + s))


def _posterior_body(x_ref, q_ref, c_ref, o_ref):
    C = x_ref.shape[0]
    nb = x_ref.shape[1]
    ca = c_ref[0, 0]
    cb = c_ref[1, 0]
    sub = min(_SUB, nb)

    for i in range(nb // sub):
        sl = slice(i * sub, (i + 1) * sub)
        x = x_ref[:, sl]
        s = jnp.sum(x, axis=0, keepdims=True)
        L = ca * s + cb * x
        den = jnp.where(L == 0.0, 1e-6, L)
        R = 1.0 / den
        for j in range(C):
            rj = jnp.broadcast_to(R[j:j + 1, :], (C, sub))
            o_ref[j, :, sl] = (q_ref[j] * L) * rj


def _run(vT, q3, coef, C, n, nb):
    grid = n // nb
    return pl.pallas_call(
        _posterior_body,
        grid=(grid,),
        in_specs=[
            pl.BlockSpec((C, nb), lambda i: (0, i)),
            pl.BlockSpec((C, C, 1), lambda i: (0, 0, 0)),
            pl.BlockSpec((2, 1), lambda i: (0, 0)),
        ],
        out_specs=pl.BlockSpec((C, C, nb), lambda i: (0, 0, i)),
        out_shape=jax.ShapeDtypeStruct((C, C, n), jnp.float32),
        name=f"discrete_noise_posterior_c{C}",
    )(vT, q3, coef)


def kernel(Xt, Et, t):
    if Et.ndim == 3:
        Et = Et.reshape((-1, Et.shape[-1]))
    n_edges = Et.shape[0]

    a = _alpha(t).astype(jnp.float32)
    ab = _alpha(t - 1).astype(jnp.float32)

    def q3_of(C):
        o = ab / C
        d = o + (1.0 - ab)
        q = jnp.full((C, C), o, jnp.float32) + (d - o) * jnp.eye(
            C, dtype=jnp.float32)
        return q[:, :, None]

    def coef_of(C):
        return jnp.stack([a / C, 1.0 - a]).astype(jnp.float32)[:, None]

    Ep = _run(Et.T, q3_of(NE), coef_of(NE), NE, n_edges, _NB)
    Xp = _run(Xt.T, q3_of(NX), coef_of(NX), NX, N_NODES, N_NODES)
    return (jnp.transpose(Xp, (2, 0, 1)), jnp.transpose(Ep, (2, 0, 1)))


# speedup vs baseline: 1.0029x; 1.0029x over previous
import jax
import jax.numpy as jnp
from jax import lax
from jax.experimental import pallas as pl

NX = 10
NE = 5
T = 1000
N_NODES = 1024

_NB = 131072
_SUB = 512


def _alpha(t):
    halfpi = 0.5 * jnp.pi
    s = 0.01
    return jnp.cos(halfpi * (t / T + s) / (1 + s))


def _posterior_body(x_ref, q_ref, c_ref, o_ref):
    C = x_ref.shape[0]
    nb = x_ref.shape[1]
    ca = c_ref[0, 0]
    cb = c_ref[1, 0]
    sub = min(_SUB, nb)

    for i in range(nb // sub):
        sl = slice(i * sub, (i + 1) * sub)
        x = x_ref[:, sl]
        s = jnp.sum(x, axis=0, keepdims=True)
        L = ca * s + cb * x
        den = jnp.where(L == 0.0, 1e-6, L)
        R = 1.0 / den
        for j in range(C):
            rj = jnp.broadcast_to(R[j:j + 1, :], (C, sub))
            o_ref[j, :, sl] = (q_ref[j] * L) * rj


def _run(vT, q3, coef, C, n, nb):
    grid = n // nb
    return pl.pallas_call(
        _posterior_body,
        grid=(grid,),
        in_specs=[
            pl.BlockSpec((C, nb), lambda i: (0, i)),
            pl.BlockSpec((C, C, 1), lambda i: (0, 0, 0)),
            pl.BlockSpec((2, 1), lambda i: (0, 0)),
        ],
        out_specs=pl.BlockSpec((C, C, nb), lambda i: (0, 0, i)),
        out_shape=jax.ShapeDtypeStruct((C, C, n), jnp.float32),
        name=f"discrete_noise_posterior_c{C}",
    )(vT, q3, coef)


def kernel(Xt, Et, t):
    if Et.ndim == 3:
        Et = Et.reshape((-1, Et.shape[-1]))
    n_edges = Et.shape[0]

    a = _alpha(t).astype(jnp.float32)
    ab = _alpha(t - 1).astype(jnp.float32)

    def q3_of(C):
        o = ab / C
        d = o + (1.0 - ab)
        q = jnp.full((C, C), o, jnp.float32) + (d - o) * jnp.eye(
            C, dtype=jnp.float32)
        return q[:, :, None]

    def coef_of(C):
        return jnp.stack([a / C, 1.0 - a]).astype(jnp.float32)[:, None]

    Ep = _run(Et.T, q3_of(NE), coef_of(NE), NE, n_edges, _NB)
    Xp = _run(Xt.T, q3_of(NX), coef_of(NX), NX, N_NODES, N_NODES)
    return (jnp.transpose(Xp, (2, 0, 1)), jnp.transpose(Ep, (2, 0, 1)))
